# baseline (speedup 1.0000x reference)
import jax
import jax.numpy as jnp
from jax import lax
from jax.experimental import pallas as pl
from jax.experimental.pallas import tpu as pltpu

_TOP_K = 50
_MIN_KEEP = 5
_EPS = 1e-5
_PENALTY = 1.1
_TOP_P = 0.8
_CAND = 64
_NEG = float("-inf")
_BIGI = 2**30
_LANES = 128


def _fold(t):
    f = t[:, 0:_LANES]
    for k in range(1, t.shape[1] // _LANES):
        f = jnp.maximum(f, t[:, k * _LANES:(k + 1) * _LANES])
    return f


def _body(ids_ref, hid_ref, gamma_ref, beta_ref, w_ref,
          probs_ref, tok_ref, h_ref, cv_ref, ci_ref, t_ref, f_ref, V):
    i = pl.program_id(0)
    nt = pl.num_programs(0)
    B, TV = t_ref.shape

    @pl.when(i == 0)
    def _init():
        x = hid_ref[...]
        mu = jnp.mean(x, axis=-1, keepdims=True)
        var = jnp.var(x, axis=-1, keepdims=True)
        h = (x - mu) / jnp.sqrt(var + _EPS)
        h_ref[...] = h * gamma_ref[...] + beta_ref[...]
        cv_ref[...] = jnp.full((B, _CAND), _NEG, jnp.float32)
        ci_ref[...] = jnp.zeros((B, _CAND), jnp.int32)

    t = lax.dot_general(h_ref[...], w_ref[...],
                        (((1,), (1,)), ((), ())),
                        preferred_element_type=jnp.float32)
    base = i * TV
    tcol = lax.broadcasted_iota(jnp.int32, (B, TV), 1)
    t = jnp.where(base + tcol < V, t, _NEG)

    F = _fold(t)
    any_ins = jnp.any(jnp.max(F, axis=1) > jnp.min(cv_ref[...], axis=1))

    @pl.when(any_ins)
    def _merge():
        t_ref[...] = t
        f_ref[...] = F
        ccol = lax.broadcasted_iota(jnp.int32, (B, _CAND), 1)
        ids = ids_ref[...]

        def cond(go):
            return go

        def body(go):
            tt = t_ref[...]
            fmax = jnp.max(f_ref[...], axis=1, keepdims=True)
            cv = cv_ref[...]
            ci = ci_ref[...]
            cmin = jnp.min(cv, axis=1, keepdims=True)
            tpos = jnp.min(jnp.where(tt == fmax, tcol, _BIGI),
                           axis=1, keepdims=True)
            ttok = base + tpos
            member = jnp.any(ids == ttok, axis=1, keepdims=True)
            pv = jnp.where(member,
                           jnp.where(fmax < 0, fmax * _PENALTY,
                                     fmax / _PENALTY),
                           fmax)
            upd = pv > cmin
            mtok = jnp.max(jnp.where(cv == cmin, ci, -1),
                           axis=1, keepdims=True)
            csel = (cv == cmin) & (ci == mtok)
            cpos = jnp.min(jnp.where(csel, ccol, _BIGI),
                           axis=1, keepdims=True)
            sel = upd & (ccol == cpos)
            cv = jnp.where(sel, pv, cv)
            ci = jnp.where(sel, ttok, ci)
            cv_ref[...] = cv
            ci_ref[...] = ci
            tt = jnp.where((fmax > cmin) & (tcol == tpos), _NEG, tt)
            t_ref[...] = tt
            F2 = _fold(tt)
            f_ref[...] = F2
            return jnp.any(jnp.max(F2, axis=1) > jnp.min(cv, axis=1))

        lax.while_loop(cond, body, any_ins)

    @pl.when(i == nt - 1)
    def _finalize():
        ccol = lax.broadcasted_iota(jnp.int32, (B, _CAND), 1)
        cv = cv_ref[...]
        ci = ci_ref[...]
        sv = jnp.full((B, _CAND), _NEG, jnp.float32)
        stok = jnp.zeros((B, _CAND), jnp.int32)
        for r in range(_TOP_K):
            m = jnp.max(cv, axis=1, keepdims=True)
            mtok = jnp.min(jnp.where(cv == m, ci, _BIGI), axis=1, keepdims=True)
            sv = jnp.where(ccol == r, m, sv)
            stok = jnp.where(ccol == r, mtok, stok)
            cv = jnp.where((cv == m) & (ci == mtok), _NEG, cv)
        mx = jnp.max(sv, axis=1, keepdims=True)
        ex = jnp.exp(sv - mx)
        p = ex / jnp.sum(ex, axis=1, keepdims=True)
        tri = (lax.broadcasted_iota(jnp.int32, (_CAND, _CAND), 0)
               <= lax.broadcasted_iota(jnp.int32, (_CAND, _CAND), 1)
               ).astype(jnp.float32)
        cum = lax.dot_general(p, tri, (((1,), (0,)), ((), ())),
                              precision=lax.Precision.HIGHEST,
                              preferred_element_type=jnp.float32)
        keepm = (cum < _TOP_P) | (ccol < _MIN_KEEP)
        filt = jnp.where(keepm, sv, jnp.float32(-1000.0))
        fmx = jnp.max(filt, axis=1, keepdims=True)
        fex = jnp.exp(filt - fmx)
        probs = fex / jnp.sum(fex, axis=1, keepdims=True)
        probs_ref[...] = probs[:, :_TOP_K]
        tok_ref[...] = stok[:, :_TOP_K]


def kernel(input_ids, hidden_states, gamma, beta, W):
    import functools
    B, D = hidden_states.shape
    V = W.shape[0]
    HIST = input_ids.shape[1]
    TV = 2048
    nt = -(-V // TV)

    in_specs = [
        pl.BlockSpec((B, HIST), lambda i: (0, 0)),
        pl.BlockSpec((B, D), lambda i: (0, 0)),
        pl.BlockSpec((1, D), lambda i: (0, 0)),
        pl.BlockSpec((1, D), lambda i: (0, 0)),
        pl.BlockSpec((TV, D), lambda i: (i, 0)),
    ]
    out_specs = [
        pl.BlockSpec((B, _TOP_K), lambda i: (0, 0)),
        pl.BlockSpec((B, _TOP_K), lambda i: (0, 0)),
    ]
    probs, token = pl.pallas_call(
        functools.partial(_body, V=V),
        grid=(nt,),
        in_specs=in_specs,
        out_specs=out_specs,
        out_shape=[
            jax.ShapeDtypeStruct((B, _TOP_K), jnp.float32),
            jax.ShapeDtypeStruct((B, _TOP_K), jnp.int32),
        ],
        scratch_shapes=[
            pltpu.VMEM((B, D), jnp.float32),
            pltpu.VMEM((B, _CAND), jnp.float32),
            pltpu.VMEM((B, _CAND), jnp.int32),
            pltpu.VMEM((B, TV), jnp.float32),
            pltpu.VMEM((B, _LANES), jnp.float32),
        ],
        compiler_params=pltpu.CompilerParams(
            dimension_semantics=("arbitrary",)),
    )(input_ids, hidden_states, gamma.reshape(1, D), beta.reshape(1, D), W)
    return probs, token
